# Initial kernel scaffold; baseline (speedup 1.0000x reference)
import jax
import jax.numpy as jnp
from jax.experimental import pallas as pl


def kernel(predictions, labels, indices, features):
    raise NotImplementedError("write your pallas kernel here")


# speedup vs baseline: 7.2046x; 7.2046x over previous
import functools

import jax
import jax.numpy as jnp
from jax import lax
from jax.experimental import pallas as pl
from jax.experimental.pallas import tpu as pltpu
from jax.experimental.pallas import tpu_sc as plsc

_H = 512
_W = 512
_NPIX = _H * _W
_NF = 512
_DS = 32
_BN = 2
_RATIO = 1.5
_THRESHOLD = 36.0


_BCE_ROWS = 384


def _bce_body(p_ref, l_ref, out_ref):
    p = p_ref[...]
    l = l_ref[...]
    lp = jnp.maximum(jnp.log(p), -100.0)
    l1p = jnp.maximum(jnp.log(1.0 - p), -100.0)
    s = -(l * lp + (1.0 - l) * l1p)

    @pl.when(pl.program_id(0) == 0)
    def _():
        out_ref[0, 0] = 0.0

    out_ref[0, 0] += jnp.sum(s)


def _semantic_sum(p2d, l2d):
    nrows = p2d.shape[0]
    grid = nrows // _BCE_ROWS
    return pl.pallas_call(
        _bce_body,
        grid=(grid,),
        in_specs=[
            pl.BlockSpec((_BCE_ROWS, _W), lambda i: (i, 0)),
            pl.BlockSpec((_BCE_ROWS, _W), lambda i: (i, 0)),
        ],
        out_specs=pl.BlockSpec(memory_space=pltpu.SMEM, block_shape=(1, 1),
                               index_map=lambda i: (0, 0)),
        out_shape=jax.ShapeDtypeStruct((1, 1), jnp.float32),
    )(p2d, l2d)


def _ham_body(ori_ref, oth_ref, w_ref, sel_ref):
    a = ori_ref[0]
    b = oth_ref[0, 0]

    def bits(x):
        planes = [((x >> k) & 1).astype(jnp.float32) for k in range(8)]
        return jnp.concatenate(planes, axis=0)

    ba = bits(a)
    bb = bits(b)
    rsa = jnp.sum(ba, axis=0)
    rsb = jnp.sum(bb, axis=0)
    m = lax.dot_general(bb, ba, (((0,), (0,)), ((), ())),
                        preferred_element_type=jnp.float32)
    d = rsb[:, None] + rsa[None, :] - 2.0 * m
    val1 = jnp.min(d, axis=1)
    eq = d == val1[:, None]
    iota = lax.broadcasted_iota(jnp.int32, (_NF, _NF), 1)
    idx1 = jnp.min(jnp.where(eq, iota, _NF), axis=1)
    cnt = jnp.sum(jnp.where(eq, 1, 0), axis=1)
    rest = jnp.min(jnp.where(eq, jnp.float32(1e9), d), axis=1)
    val2 = jnp.where(cnt >= 2, val1, rest)
    w = (val1 < _RATIO * val2).astype(jnp.float32)
    w_ref[0, 0, 0, :] = w
    sel_ref[0, 0, 0, :] = idx1


def _hamming_top2(ori, oth):
    return pl.pallas_call(
        _ham_body,
        grid=(_BN, 2),
        in_specs=[
            pl.BlockSpec((1, _DS, _NF), lambda b, r: (b, 0, 0)),
            pl.BlockSpec((1, 1, _DS, _NF), lambda b, r: (r, b, 0, 0)),
        ],
        out_specs=[
            pl.BlockSpec((1, 1, 1, _NF), lambda b, r: (r, b, 0, 0)),
            pl.BlockSpec((1, 1, 1, _NF), lambda b, r: (r, b, 0, 0)),
        ],
        out_shape=[
            jax.ShapeDtypeStruct((2, _BN, 1, _NF), jnp.float32),
            jax.ShapeDtypeStruct((2, _BN, 1, _NF), jnp.int32),
        ],
    )(ori, oth)


_CHUNK = 128
_NCHUNK = _NF // _CHUNK


def _sc_gather_body(preds_ref, idx_ref, sel_ref,
                    pred_ori_ref, loc_sel_ref, pred_sel_ref,
                    selv, locv, pidxv, glocv, gpredv, predv, sem):
    c = lax.axis_index("c")
    s = lax.axis_index("s")
    wid = s * 2 + c

    @pl.when(wid < 2 * _NCHUNK)
    def _direct():
        row = wid // _NCHUNK
        ch = wid % _NCHUNK
        pltpu.sync_copy(idx_ref.at[row, pl.ds(ch * _CHUNK, _CHUNK)], selv)
        base = row * _NPIX
        for j in range(_CHUNK // 16):
            gpredv[pl.ds(j * 16, 16)] = selv[pl.ds(j * 16, 16)] + base
        pltpu.async_copy(preds_ref.at[gpredv], predv, sem).wait()
        pltpu.sync_copy(predv, pred_ori_ref.at[row, pl.ds(ch * _CHUNK, _CHUNK)])

    @pl.when(jnp.logical_and(wid >= 2 * _NCHUNK, wid < 6 * _NCHUNK))
    def _branch():
        u = wid - 2 * _NCHUNK
        br = u // _NCHUNK
        ch = u % _NCHUNK
        locrow = jnp.where(br < 2, 2, 4)
        predrow = 2 + (br & 1)
        pltpu.sync_copy(idx_ref.at[locrow], locv)
        pltpu.sync_copy(idx_ref.at[predrow], pidxv)
        pltpu.sync_copy(sel_ref.at[br, pl.ds(ch * _CHUNK, _CHUNK)], selv)
        base = predrow * _NPIX
        for j in range(_CHUNK // 16):
            sv = selv[pl.ds(j * 16, 16)]
            glocv[pl.ds(j * 16, 16)] = plsc.load_gather(locv, [sv])
            gpredv[pl.ds(j * 16, 16)] = plsc.load_gather(pidxv, [sv]) + base
        pltpu.sync_copy(glocv, loc_sel_ref.at[br, pl.ds(ch * _CHUNK, _CHUNK)])
        pltpu.async_copy(preds_ref.at[gpredv], predv, sem).wait()
        pltpu.sync_copy(predv, pred_sel_ref.at[br, pl.ds(ch * _CHUNK, _CHUNK)])


def _sc_gather(preds4, idx, sel4):
    mesh = plsc.VectorSubcoreMesh(core_axis_name="c", subcore_axis_name="s")
    fn = pl.kernel(
        _sc_gather_body,
        mesh=mesh,
        compiler_params=pltpu.CompilerParams(needs_layout_passes=False),
        out_type=[
            jax.ShapeDtypeStruct((_BN, _NF), jnp.float32),
            jax.ShapeDtypeStruct((4, _NF), jnp.int32),
            jax.ShapeDtypeStruct((4, _NF), jnp.float32),
        ],
        scratch_types=[
            pltpu.VMEM((_CHUNK,), jnp.int32),
            pltpu.VMEM((_NF,), jnp.int32),
            pltpu.VMEM((_NF,), jnp.int32),
            pltpu.VMEM((_CHUNK,), jnp.int32),
            pltpu.VMEM((_CHUNK,), jnp.int32),
            pltpu.VMEM((_CHUNK,), jnp.float32),
            pltpu.SemaphoreType.DMA,
        ],
    )
    return fn(preds4, idx, sel4)


def _branch_body(ls_ref, lo_ref, ps_ref, po_ref, w_ref, out_ref):
    ls = ls_ref[0, 0, :]
    lo = lo_ref[0, 0, :]
    ps = ps_ref[0, 0, :]
    po = po_ref[0, 0, :]
    w = w_ref[0, 0, :]
    count = jnp.sum(w)
    xs = (ls >> 9).astype(jnp.float32)
    ys = (ls & (_W - 1)).astype(jnp.float32)
    xo = (lo >> 9).astype(jnp.float32)
    yo = (lo & (_W - 1)).astype(jnp.float32)
    mxs = jnp.sum(xs * w) / count
    mys = jnp.sum(ys * w) / count
    mxo = jnp.sum(xo * w) / count
    myo = jnp.sum(yo * w) / count
    xn = (xs - mxs) * w
    yn = (ys - mys) * w
    xon = (xo - mxo) * w
    yon = (yo - myo) * w
    z = jnp.zeros((_NF,), jnp.float32)
    o = jnp.ones((_NF,), jnp.float32)
    r1 = jnp.stack([xon, yon, o, z, z, z, -xon * xn, -yon * xn], axis=-1)
    r1 = r1 * w[:, None]
    r2 = jnp.stack([z, z, z, xon, yon, o, -xon * yn, -yon * yn], axis=-1)
    r2 = r2 * w[:, None]
    dn = (((0,), (0,)), ((), ()))
    g = (lax.dot_general(r1, r1, dn, preferred_element_type=jnp.float32,
                         precision=lax.Precision.HIGHEST)
         + lax.dot_general(r2, r2, dn, preferred_element_type=jnp.float32,
                           precision=lax.Precision.HIGHEST))
    b1 = (xn * w)[:, None]
    b2 = (yn * w)[:, None]
    cvec = (lax.dot_general(r1, b1, dn, preferred_element_type=jnp.float32,
                            precision=lax.Precision.HIGHEST)
            + lax.dot_general(r2, b2, dn, preferred_element_type=jnp.float32,
                              precision=lax.Precision.HIGHEST))
    a = jnp.concatenate([g, cvec], axis=1)
    rows = lax.broadcasted_iota(jnp.int32, (8, 1), 0)
    for k in range(8):
        piv = a[k, k]
        fac = a[:, k:k + 1] / piv
        rowk = a[k:k + 1, :]
        mask = rows == k
        a = a - jnp.where(mask, 0.0, fac) * rowk
        a = jnp.where(mask, a / piv, a)
    h = a[:, 8]
    s0 = h[0] * xon + h[1] * yon + h[2]
    s1 = h[3] * xon + h[4] * yon + h[5]
    s2 = h[6] * xon + h[7] * yon + 1.0
    d = jnp.sqrt((xn - s0 / s2) ** 2 + (yn - s1 / s2) ** 2)
    out_ref[pl.program_id(0), 0] = jnp.sum(w * d * po * ps) / count


def _branches(loc_sel, loc_ori, pred_sel, pred_ori, w4):
    return pl.pallas_call(
        _branch_body,
        grid=(4,),
        in_specs=[
            pl.BlockSpec((1, 1, _NF), lambda i: (i, 0, 0)),
            pl.BlockSpec((1, 1, _NF), lambda i: (i, 0, 0)),
            pl.BlockSpec((1, 1, _NF), lambda i: (i, 0, 0)),
            pl.BlockSpec((1, 1, _NF), lambda i: (i, 0, 0)),
            pl.BlockSpec((1, 1, _NF), lambda i: (i, 0, 0)),
        ],
        out_specs=pl.BlockSpec(memory_space=pltpu.SMEM, block_shape=(4, 1),
                               index_map=lambda i: (0, 0)),
        out_shape=jax.ShapeDtypeStruct((4, 1), jnp.float32),
    )(loc_sel, loc_ori, pred_sel, pred_ori, w4)


def kernel(predictions, labels, indices, features):
    b3 = features.shape[0]
    p2d = predictions.reshape(b3 * _H, _W)
    l2d = labels.reshape(b3 * _H, _W)
    sem_sum = _semantic_sum(p2d, l2d)
    semantic = sem_sum[0, 0] / jnp.float32(b3 * _NPIX)

    ori = features[0:_BN]
    oth = features[_BN:].reshape(2, _BN, _DS, _NF)
    w4_raw, sel4_raw = _hamming_top2(ori, oth)
    w4 = w4_raw.reshape(4, 1, _NF)
    sel4 = sel4_raw.reshape(4, _NF)

    idx = indices[:, 0, :, 0]
    preds4 = predictions.reshape(b3, _NPIX)[0:4].reshape(-1)
    pred_ori2, loc_sel4, pred_sel4 = _sc_gather(preds4, idx, sel4)

    loc_ori4 = jnp.concatenate([idx[0:2], idx[0:2]], axis=0).reshape(4, 1, _NF)
    pred_ori4 = jnp.concatenate([pred_ori2, pred_ori2], axis=0).reshape(4, 1, _NF)
    res = _branches(loc_sel4.reshape(4, 1, _NF), loc_ori4,
                    pred_sel4.reshape(4, 1, _NF), pred_ori4, w4)

    dp = res[0, 0] + res[1, 0]
    dn = res[2, 0] + res[3, 0]
    triplet = jnp.maximum(dp - dn + _THRESHOLD, 0.0) / jnp.float32(_BN)
    return semantic + triplet


# speedup vs baseline: 9.2666x; 1.2862x over previous
import functools

import jax
import jax.numpy as jnp
from jax import lax
from jax.experimental import pallas as pl
from jax.experimental.pallas import tpu as pltpu
from jax.experimental.pallas import tpu_sc as plsc

_H = 512
_W = 512
_NPIX = _H * _W
_NF = 512
_DS = 32
_BN = 2
_B3 = 6
_RATIO = 1.5
_THRESHOLD = 36.0


_BCE_ROWS = _B3 * _H // 4


def _ham_bce_body(ori_ref, oth_ref, p_ref, l_ref, w_ref, sel_ref, bce_ref):
    p = p_ref[...]
    l = l_ref[...]
    lp = jnp.maximum(jnp.log(p), -100.0)
    l1p = jnp.maximum(jnp.log(1.0 - p), -100.0)
    s = -(l * lp + (1.0 - l) * l1p)

    @pl.when(jnp.logical_and(pl.program_id(0) == 0, pl.program_id(1) == 0))
    def _():
        bce_ref[0, 0] = 0.0

    bce_ref[0, 0] += jnp.sum(s)

    a = ori_ref[0]
    b = oth_ref[0, 0]

    def bits(x):
        planes = [((x >> k) & 1).astype(jnp.float32) for k in range(8)]
        return jnp.concatenate(planes, axis=0)

    ba = bits(a)
    bb = bits(b)
    rsa = jnp.sum(ba, axis=0)
    rsb = jnp.sum(bb, axis=0)
    m = lax.dot_general(bb, ba, (((0,), (0,)), ((), ())),
                        preferred_element_type=jnp.float32)
    d = rsb[:, None] + rsa[None, :] - 2.0 * m
    val1 = jnp.min(d, axis=1)
    eq = d == val1[:, None]
    iota = lax.broadcasted_iota(jnp.int32, (_NF, _NF), 1)
    idx1 = jnp.min(jnp.where(eq, iota, _NF), axis=1)
    cnt = jnp.sum(jnp.where(eq, 1, 0), axis=1)
    rest = jnp.min(jnp.where(eq, jnp.float32(1e9), d), axis=1)
    val2 = jnp.where(cnt >= 2, val1, rest)
    w = (val1 < _RATIO * val2).astype(jnp.float32)
    w_ref[0, 0, 0, :] = w
    sel_ref[0, 0, 0, :] = idx1


def _ham_bce(ori, oth, p2d, l2d):
    return pl.pallas_call(
        _ham_bce_body,
        grid=(_BN, 2),
        in_specs=[
            pl.BlockSpec((1, _DS, _NF), lambda b, r: (b, 0, 0)),
            pl.BlockSpec((1, 1, _DS, _NF), lambda b, r: (r, b, 0, 0)),
            pl.BlockSpec((_BCE_ROWS, _W), lambda b, r: (b * 2 + r, 0)),
            pl.BlockSpec((_BCE_ROWS, _W), lambda b, r: (b * 2 + r, 0)),
        ],
        out_specs=[
            pl.BlockSpec((1, 1, 1, _NF), lambda b, r: (r, b, 0, 0)),
            pl.BlockSpec((1, 1, 1, _NF), lambda b, r: (r, b, 0, 0)),
            pl.BlockSpec(memory_space=pltpu.SMEM, block_shape=(1, 1),
                         index_map=lambda b, r: (0, 0)),
        ],
        out_shape=[
            jax.ShapeDtypeStruct((2, _BN, 1, _NF), jnp.float32),
            jax.ShapeDtypeStruct((2, _BN, 1, _NF), jnp.int32),
            jax.ShapeDtypeStruct((1, 1), jnp.float32),
        ],
    )(ori, oth, p2d, l2d)


_CHUNK = 64


def _sc_gather_body(preds_ref, idx_ref, out_ref, selv, gpredv, predv, sem):
    c = lax.axis_index("c")
    s = lax.axis_index("s")
    wid = s * 2 + c
    row = wid // 8
    ch = wid % 8
    pltpu.sync_copy(idx_ref.at[row, pl.ds(ch * _CHUNK, _CHUNK)], selv)
    base = row * _NPIX
    for j in range(_CHUNK // 16):
        gpredv[pl.ds(j * 16, 16)] = selv[pl.ds(j * 16, 16)] + base
    pltpu.async_copy(preds_ref.at[gpredv], predv, sem).wait()
    pltpu.sync_copy(predv, out_ref.at[row, pl.ds(ch * _CHUNK, _CHUNK)])


def _sc_gather(preds4, idx):
    mesh = plsc.VectorSubcoreMesh(core_axis_name="c", subcore_axis_name="s")
    fn = pl.kernel(
        _sc_gather_body,
        mesh=mesh,
        compiler_params=pltpu.CompilerParams(needs_layout_passes=False),
        out_type=jax.ShapeDtypeStruct((4, _NF), jnp.float32),
        scratch_types=[
            pltpu.VMEM((_CHUNK,), jnp.int32),
            pltpu.VMEM((_CHUNK,), jnp.int32),
            pltpu.VMEM((_CHUNK,), jnp.float32),
            pltpu.SemaphoreType.DMA,
        ],
    )
    return fn(preds4, idx)


def _branch_body(sem_ref, idx_ref, sel_ref, w_ref, p4_ref, out_ref):
    sel = sel_ref[:, 0, :]
    w = w_ref[:, 0, :]
    lo = jnp.concatenate([idx_ref[0:2, 0, :]] * 2, axis=0)
    locv = jnp.concatenate([idx_ref[2:3, 0, :], idx_ref[2:3, 0, :],
                            idx_ref[4:5, 0, :], idx_ref[4:5, 0, :]], axis=0)
    pvv = jnp.concatenate([p4_ref[2:4, 0, :]] * 2, axis=0)
    po = jnp.concatenate([p4_ref[0:2, 0, :]] * 2, axis=0)
    iota = lax.broadcasted_iota(jnp.int32, (4, _NF, _NF), 2)
    eq = iota == sel[:, :, None]
    locg = jnp.min(jnp.where(eq, locv[:, None, :], jnp.int32(2 ** 30)), axis=2)
    ps = jnp.min(jnp.where(eq, pvv[:, None, :], jnp.float32(1e30)), axis=2)
    xs = (locg >> 9).astype(jnp.float32)
    ys = (locg & (_W - 1)).astype(jnp.float32)
    xo = (lo >> 9).astype(jnp.float32)
    yo = (lo & (_W - 1)).astype(jnp.float32)
    count = jnp.sum(w, axis=1, keepdims=True)
    mxs = jnp.sum(xs * w, axis=1, keepdims=True) / count
    mys = jnp.sum(ys * w, axis=1, keepdims=True) / count
    mxo = jnp.sum(xo * w, axis=1, keepdims=True) / count
    myo = jnp.sum(yo * w, axis=1, keepdims=True) / count
    xn = (xs - mxs) * w
    yn = (ys - mys) * w
    xon = (xo - mxo) * w
    yon = (yo - myo) * w
    z = jnp.zeros((4, _NF), jnp.float32)
    o = jnp.ones((4, _NF), jnp.float32)
    r1 = jnp.stack([xon, yon, o, z, z, z, -xon * xn, -yon * xn], axis=-1)
    r1 = r1 * w[:, :, None]
    r2 = jnp.stack([z, z, z, xon, yon, o, -xon * yn, -yon * yn], axis=-1)
    r2 = r2 * w[:, :, None]
    bnum = (((1,), (1,)), ((0,), (0,)))
    g8 = (lax.dot_general(r1, r1, bnum, preferred_element_type=jnp.float32,
                          precision=lax.Precision.HIGHEST)
          + lax.dot_general(r2, r2, bnum, preferred_element_type=jnp.float32,
                            precision=lax.Precision.HIGHEST))
    b1 = (xn * w)[:, :, None]
    b2 = (yn * w)[:, :, None]
    cvec = (lax.dot_general(r1, b1, bnum, preferred_element_type=jnp.float32,
                            precision=lax.Precision.HIGHEST)
            + lax.dot_general(r2, b2, bnum, preferred_element_type=jnp.float32,
                              precision=lax.Precision.HIGHEST))
    a = jnp.concatenate([g8, cvec], axis=2)
    rows8 = lax.broadcasted_iota(jnp.int32, (4, 8, 1), 1)
    for k in range(8):
        piv = a[:, k:k + 1, k:k + 1]
        fac = a[:, :, k:k + 1] / piv
        rowk = a[:, k:k + 1, :]
        mask = rows8 == k
        a = a - jnp.where(mask, 0.0, fac) * rowk
        a = jnp.where(mask, a / piv, a)
    h = a[:, :, 8]
    s0 = h[:, 0:1] * xon + h[:, 1:2] * yon + h[:, 2:3]
    s1 = h[:, 3:4] * xon + h[:, 4:5] * yon + h[:, 5:6]
    s2 = h[:, 6:7] * xon + h[:, 7:8] * yon + 1.0
    d = jnp.sqrt((xn - s0 / s2) ** 2 + (yn - s1 / s2) ** 2)
    res = jnp.sum(w * d * po * ps, axis=1) / count[:, 0]
    dp = res[0] + res[1]
    dn = res[2] + res[3]
    triplet = jnp.maximum(dp - dn + _THRESHOLD, 0.0) / jnp.float32(_BN)
    out_ref[0, 0] = sem_ref[0, 0] / jnp.float32(_B3 * _NPIX) + triplet


def _branches(sem, idx6, sel4, w4, p4):
    return pl.pallas_call(
        _branch_body,
        grid=(1,),
        in_specs=[
            pl.BlockSpec(memory_space=pltpu.SMEM, block_shape=(1, 1),
                         index_map=lambda i: (0, 0)),
            pl.BlockSpec((_B3, 1, _NF), lambda i: (0, 0, 0)),
            pl.BlockSpec((4, 1, _NF), lambda i: (0, 0, 0)),
            pl.BlockSpec((4, 1, _NF), lambda i: (0, 0, 0)),
            pl.BlockSpec((4, 1, _NF), lambda i: (0, 0, 0)),
        ],
        out_specs=pl.BlockSpec(memory_space=pltpu.SMEM, block_shape=(1, 1),
                               index_map=lambda i: (0, 0)),
        out_shape=jax.ShapeDtypeStruct((1, 1), jnp.float32),
    )(sem, idx6, sel4, w4, p4)


def kernel(predictions, labels, indices, features):
    p2d = predictions.reshape(_B3 * _H, _W)
    l2d = labels.reshape(_B3 * _H, _W)
    ori = features[0:_BN]
    oth = features[_BN:].reshape(2, _BN, _DS, _NF)
    w4_raw, sel4_raw, sem_sum = _ham_bce(ori, oth, p2d, l2d)

    idx = indices[:, 0, :, 0]
    preds4 = predictions.reshape(_B3, _NPIX)[0:4].reshape(-1)
    p4 = _sc_gather(preds4, idx)

    res = _branches(sem_sum, idx.reshape(_B3, 1, _NF),
                    sel4_raw.reshape(4, 1, _NF), w4_raw.reshape(4, 1, _NF),
                    p4.reshape(4, 1, _NF))
    return res[0, 0]
